# baseline (speedup 1.0000x reference)
import functools

import jax
import jax.numpy as jnp
from jax import lax
from jax.experimental import pallas as pl
from jax.experimental.pallas import tpu as pltpu
from jax.experimental.pallas import tpu_sc as plsc

B = 4
C = 96
HW = 512 * 512
NW = 32
GROUPS = NW // B
CPG = C // GROUPS
LANES = 16
CHUNK = 2048
NCHUNK = HW // CHUNK
NACC = 1 + 2 * CPG


def _sc_partials(feat2, mask2):
    mesh = plsc.VectorSubcoreMesh(core_axis_name="c", subcore_axis_name="s")

    @functools.partial(
        pl.kernel,
        mesh=mesh,
        out_type=jax.ShapeDtypeStruct((NW, NACC, LANES), jnp.float32),
        scratch_types=[
            pltpu.VMEM((2, CPG, CHUNK), jnp.float32),
            pltpu.VMEM((2, CHUNK), jnp.float32),
            pltpu.VMEM((NACC, LANES), jnp.float32),
            pltpu.SemaphoreType.DMA,
            pltpu.SemaphoreType.DMA,
        ],
    )
    def kern(feat_h, mask_h, out_h, fbuf, mbuf, obuf, sem0, sem1):
        wid = lax.axis_index("s") * 2 + lax.axis_index("c")
        b = wid // GROUPS
        ch0 = (wid % GROUPS) * CPG
        sems = (sem0, sem1)

        def dma_descs(slot, j):
            off = j * CHUNK
            ds = [
                pltpu.make_async_copy(
                    mask_h.at[b, pl.ds(off, CHUNK)], mbuf.at[slot], sems[slot]
                )
            ]
            for c in range(CPG):
                ds.append(
                    pltpu.make_async_copy(
                        feat_h.at[b, ch0 + c, pl.ds(off, CHUNK)],
                        fbuf.at[slot, c],
                        sems[slot],
                    )
                )
            return ds

        def issue(slot, j):
            for d in dma_descs(slot, j):
                d.start()

        def drain(slot, j):
            for d in dma_descs(slot, j):
                d.wait()

        UNROLL = 8

        def compute_slot(slot, acc):
            def inner(i, acc):
                base0 = i * (LANES * UNROLL)
                for u in range(UNROLL):
                    cnt = acc[0]
                    base = pl.multiple_of(base0 + u * LANES, LANES)
                    m = mbuf[slot, pl.ds(base, LANES)]
                    indf = jnp.where(m > 0.5, 1.0, 0.0)
                    cnt = cnt + indf
                    new_fg = []
                    new_tot = []
                    for c in range(CPG):
                        f = fbuf[slot, c, pl.ds(base, LANES)]
                        new_fg.append(acc[1 + c] + f * indf)
                        new_tot.append(acc[1 + CPG + c] + f)
                    acc = (cnt, *new_fg, *new_tot)
                return acc

            return lax.fori_loop(0, CHUNK // (LANES * UNROLL), inner, acc)

        zero = jnp.zeros((LANES,), jnp.float32)
        acc = (zero,) * NACC

        issue(0, 0)

        def body(k, acc):
            j1 = 2 * k + 1
            issue(1, j1)
            drain(0, 2 * k)
            acc = compute_slot(0, acc)
            j2 = lax.rem(2 * k + 2, NCHUNK)
            issue(0, j2)
            drain(1, j1)
            acc = compute_slot(1, acc)
            return acc

        acc = lax.fori_loop(0, NCHUNK // 2, body, acc)
        drain(0, 0)

        for r in range(NACC):
            obuf[r, :] = acc[r]
        pltpu.sync_copy(obuf, out_h.at[wid])

    return kern(feat2, mask2)


def kernel(feat_hw, mask_hw, pf, pb, init_f, init_b):
    m = 0.9
    feat2 = feat_hw.reshape(B, C, HW)
    mask2 = mask_hw.reshape(B, HW)

    part = _sc_partials(feat2, mask2).sum(-1)
    part = part.reshape(B, GROUPS, NACC)

    cnt_fg = part[:, 0, 0].sum()
    cnt_bg = jnp.float32(B * HW) - cnt_fg
    s_fg = part[:, :, 1 : 1 + CPG].reshape(B, C).sum(0)
    s_tot = part[:, :, 1 + CPG :].reshape(B, C).sum(0)

    mf = s_fg / jnp.maximum(cnt_fg, 1.0)
    mb = (s_tot - s_fg) / jnp.maximum(cnt_bg, 1.0)

    pf_upd = jnp.where(init_f == 0, mf, pf * m + mf * (1.0 - m))
    pf_new = jnp.where(cnt_fg > 0, pf_upd, pf)
    pb_upd = jnp.where(init_b == 0, mb, pb * m + mb * (1.0 - m))
    pb_new = jnp.where(cnt_bg > 0, pb_upd, pb)
    return jnp.stack([pf_new, pb_new])


# speedup vs baseline: 3.7221x; 3.7221x over previous
import functools

import jax
import jax.numpy as jnp
from jax import lax
from jax.experimental import pallas as pl
from jax.experimental.pallas import tpu as pltpu
from jax.experimental.pallas import tpu_sc as plsc

B = 4
C = 96
H = 512
W = 512
HW = H * W
NW = 32
GROUPS = NW // B
CPG = C // GROUPS
LANES = 16
CROWS = 8
CCOLS = 256
NCH = H // CROWS
NCW = W // CCOLS
NCHUNK = NCH * NCW
NACC = 1 + 2 * CPG


def _sc_partials(feat4, mask3):
    mesh = plsc.VectorSubcoreMesh(core_axis_name="c", subcore_axis_name="s")

    @functools.partial(
        pl.kernel,
        mesh=mesh,
        out_type=jax.ShapeDtypeStruct((NW, NACC, LANES), jnp.float32),
        scratch_types=[
            pltpu.VMEM((2, CPG, CROWS, CCOLS), jnp.float32),
            pltpu.VMEM((2, CROWS, CCOLS), jnp.float32),
            pltpu.VMEM((NACC, LANES), jnp.float32),
            pltpu.SemaphoreType.DMA,
            pltpu.SemaphoreType.DMA,
        ],
        compiler_params=pltpu.CompilerParams(use_tc_tiling_on_sc=True),
    )
    def kern(feat_h, mask_h, out_h, fbuf, mbuf, obuf, sem0, sem1):
        wid = lax.axis_index("s") * 2 + lax.axis_index("c")
        b = wid // GROUPS
        ch0 = (wid % GROUPS) * CPG
        sems = (sem0, sem1)

        def dma_descs(slot, j):
            r0 = (j // NCW) * CROWS
            w0 = (j % NCW) * CCOLS
            ds = [
                pltpu.make_async_copy(
                    mask_h.at[b, pl.ds(r0, CROWS), pl.ds(w0, CCOLS)],
                    mbuf.at[slot],
                    sems[slot],
                )
            ]
            for c in range(CPG):
                ds.append(
                    pltpu.make_async_copy(
                        feat_h.at[b, ch0 + c, pl.ds(r0, CROWS), pl.ds(w0, CCOLS)],
                        fbuf.at[slot, c],
                        sems[slot],
                    )
                )
            return ds

        def issue(slot, j):
            for d in dma_descs(slot, j):
                d.start()

        def drain(slot, j):
            for d in dma_descs(slot, j):
                d.wait()

        def compute_slot(slot, acc):
            def inner(i, acc):
                base = pl.multiple_of(i * LANES, LANES)
                for r in range(CROWS):
                    cnt = acc[0]
                    m = mbuf[slot, r, pl.ds(base, LANES)]
                    indf = jnp.where(m > 0.5, 1.0, 0.0)
                    cnt = cnt + indf
                    new_fg = []
                    new_tot = []
                    for c in range(CPG):
                        f = fbuf[slot, c, r, pl.ds(base, LANES)]
                        new_fg.append(acc[1 + c] + f * indf)
                        new_tot.append(acc[1 + CPG + c] + f)
                    acc = (cnt, *new_fg, *new_tot)
                return acc

            return lax.fori_loop(0, CCOLS // LANES, inner, acc)

        zero = jnp.zeros((LANES,), jnp.float32)
        acc = (zero,) * NACC

        issue(0, 0)

        def body(k, acc):
            j1 = 2 * k + 1
            issue(1, j1)
            drain(0, 2 * k)
            acc = compute_slot(0, acc)
            j2 = lax.rem(2 * k + 2, NCHUNK)
            issue(0, j2)
            drain(1, j1)
            acc = compute_slot(1, acc)
            return acc

        acc = lax.fori_loop(0, NCHUNK // 2, body, acc)
        drain(0, 0)

        for r in range(NACC):
            obuf[r, :] = acc[r]
        pltpu.sync_copy(obuf, out_h.at[wid])

    return kern(feat4, mask3)


def kernel(feat_hw, mask_hw, pf, pb, init_f, init_b):
    m = 0.9

    part = _sc_partials(feat_hw, mask_hw).sum(-1)
    part = part.reshape(B, GROUPS, NACC)

    cnt_fg = part[:, 0, 0].sum()
    cnt_bg = jnp.float32(B * HW) - cnt_fg
    s_fg = part[:, :, 1 : 1 + CPG].reshape(B, C).sum(0)
    s_tot = part[:, :, 1 + CPG :].reshape(B, C).sum(0)

    mf = s_fg / jnp.maximum(cnt_fg, 1.0)
    mb = (s_tot - s_fg) / jnp.maximum(cnt_bg, 1.0)

    pf_upd = jnp.where(init_f == 0, mf, pf * m + mf * (1.0 - m))
    pf_new = jnp.where(cnt_fg > 0, pf_upd, pf)
    pb_upd = jnp.where(init_b == 0, mb, pb * m + mb * (1.0 - m))
    pb_new = jnp.where(cnt_bg > 0, pb_upd, pb)
    return jnp.stack([pf_new, pb_new])
